# Initial kernel scaffold; baseline (speedup 1.0000x reference)
import jax
import jax.numpy as jnp
from jax import lax
from jax.experimental import pallas as pl
from jax.experimental.pallas import tpu as pltpu
from jax.experimental.pallas import tpu_sc as plsc

N = 10000
NP = 10240
F = 128
E = 320000
NC, NS = 2, 16
NW = NC * NS
EPW = E // NW
CHUNK = 80
NCH = EPW // CHUNK
STRIPE = NP // NS
R = 1024
GRID = NP // R


def _fill_f32(ref, n, value):
    v = jnp.full((16,), value, dtype=jnp.float32)

    def body(i, _):
        ref[pl.ds(i * 16, 16)] = v
        return 0

    lax.fori_loop(0, n // 16, body, 0)


def _fill2_f32(ref, rows, value):
    v = jnp.full((16,), value, dtype=jnp.float32)

    def body(i, _):
        for j in range(8):
            ref[i, pl.ds(j * 16, 16)] = v
        return 0

    lax.fori_loop(0, rows, body, 0)


def _hist_body(edge_hbm, cnt_hbm, hist_sh, idx_v, ones_v, tmp_v):
    c = lax.axis_index("c")
    s = lax.axis_index("s")
    wid = c * NS + s

    _fill_f32(tmp_v, STRIPE, 0.0)
    pltpu.sync_copy(tmp_v, hist_sh.at[pl.ds(s * STRIPE, STRIPE)])
    _fill_f32(ones_v, CHUNK, 1.0)
    plsc.subcore_barrier()

    def body(i, _):
        base = wid * EPW + i * CHUNK
        pltpu.sync_copy(edge_hbm.at[1, pl.ds(base, CHUNK)], idx_v)
        pltpu.sync_copy(ones_v, hist_sh.at[idx_v], add=True)
        return 0

    lax.fori_loop(0, NCH, body, 0)
    plsc.subcore_barrier()

    pltpu.sync_copy(hist_sh.at[pl.ds(s * STRIPE, STRIPE)], tmp_v)
    pltpu.sync_copy(tmp_v, cnt_hbm.at[c, pl.ds(s * STRIPE, STRIPE)])


_hist = pl.kernel(
    _hist_body,
    out_type=jax.ShapeDtypeStruct((NC, NP), jnp.float32),
    mesh=plsc.VectorSubcoreMesh(core_axis_name="c", subcore_axis_name="s"),
    scratch_types=[
        pltpu.VMEM_SHARED((NP,), jnp.float32),
        pltpu.VMEM((CHUNK,), jnp.int32),
        pltpu.VMEM((CHUNK,), jnp.float32),
        pltpu.VMEM((STRIPE,), jnp.float32),
    ],
)


def _edge_body(edge_hbm, h2_hbm, s_hbm, acc_sh, ridx_v, cidx_v, rows_v, sem):
    c = lax.axis_index("c")
    s = lax.axis_index("s")
    wid = c * NS + s

    _fill2_f32(rows_v, CHUNK, 0.0)
    for j in range(STRIPE // CHUNK):
        pltpu.sync_copy(rows_v, acc_sh.at[pl.ds(s * STRIPE + j * CHUNK, CHUNK)])
    plsc.subcore_barrier()

    def body(i, _):
        base = wid * EPW + i * CHUNK
        pltpu.sync_copy(edge_hbm.at[0, pl.ds(base, CHUNK)], ridx_v)
        pltpu.async_copy(h2_hbm.at[ridx_v], rows_v, sem).wait()
        pltpu.sync_copy(edge_hbm.at[1, pl.ds(base, CHUNK)], cidx_v)
        pltpu.sync_copy(rows_v, acc_sh.at[cidx_v], add=True)
        return 0

    lax.fori_loop(0, NCH, body, 0)
    plsc.subcore_barrier()

    for j in range(STRIPE // CHUNK):
        r0 = s * STRIPE + j * CHUNK
        pltpu.sync_copy(acc_sh.at[pl.ds(r0, CHUNK)], rows_v)
        pltpu.sync_copy(rows_v, s_hbm.at[c, pl.ds(r0, CHUNK)])


_edges = pl.kernel(
    _edge_body,
    out_type=jax.ShapeDtypeStruct((NC, NP, F), jnp.float32),
    mesh=plsc.VectorSubcoreMesh(core_axis_name="c", subcore_axis_name="s"),
    scratch_types=[
        pltpu.VMEM_SHARED((NP, F), jnp.float32),
        pltpu.VMEM((CHUNK,), jnp.int32),
        pltpu.VMEM((CHUNK,), jnp.int32),
        pltpu.VMEM((CHUNK, F), jnp.float32),
        pltpu.SemaphoreType.DMA,
    ],
)


def _eye128():
    a = lax.broadcasted_iota(jnp.int32, (F, F), 0)
    b = lax.broadcasted_iota(jnp.int32, (F, F), 1)
    return (a == b).astype(jnp.float32)


def _ln_mm_body(x_ref, g_ref, bt_ref, w_ref, cnt_ref, o_ref):
    xb = x_ref[...]
    mean = jnp.mean(xb, axis=1, keepdims=True)
    xc = xb - mean
    var = jnp.mean(xc * xc, axis=1, keepdims=True)
    h = xc * lax.rsqrt(var + 1e-5) * g_ref[...] + bt_ref[...]
    h = jnp.maximum(h, 0.0)
    hw = jnp.dot(h, w_ref[...], preferred_element_type=jnp.float32)
    cb = cnt_ref[...]
    dis = lax.rsqrt(cb[0] + cb[1] + 1.0)
    eye = _eye128()
    for r in range(R // F):
        diag = eye * dis[r][None, :]
        o_ref[r * F:(r + 1) * F, :] = jnp.dot(
            diag, hw[r * F:(r + 1) * F, :], preferred_element_type=jnp.float32)


def _final_body(s_ref, h2_ref, cnt_ref, b_ref, o_ref):
    sb = s_ref[...]
    t = sb[0] + sb[1] + h2_ref[...]
    cb = cnt_ref[...]
    dis = lax.rsqrt(cb[0] + cb[1] + 1.0)
    eye = _eye128()
    bias = b_ref[...]
    for r in range(R // F):
        diag = eye * dis[r][None, :]
        o_ref[r * F:(r + 1) * F, :] = jnp.dot(
            diag, t[r * F:(r + 1) * F, :], preferred_element_type=jnp.float32) + bias


@jax.jit
def kernel(x, edge_index, gamma, beta, W, b):
    edge_index = edge_index.astype(jnp.int32)
    cnt = _hist(edge_index)

    xp = jnp.zeros((NP, F), x.dtype).at[:N].set(x)
    cnt3 = cnt.reshape(NC, NP // F, F)

    h2 = pl.pallas_call(
        _ln_mm_body,
        grid=(GRID,),
        in_specs=[
            pl.BlockSpec((R, F), lambda i: (i, 0)),
            pl.BlockSpec((1, F), lambda i: (0, 0)),
            pl.BlockSpec((1, F), lambda i: (0, 0)),
            pl.BlockSpec((F, F), lambda i: (0, 0)),
            pl.BlockSpec((NC, R // F, F), lambda i: (0, i, 0)),
        ],
        out_specs=pl.BlockSpec((R, F), lambda i: (i, 0)),
        out_shape=jax.ShapeDtypeStruct((NP, F), jnp.float32),
    )(xp, gamma.reshape(1, F), beta.reshape(1, F), W, cnt3)

    s_part = _edges(edge_index, h2)

    outp = pl.pallas_call(
        _final_body,
        grid=(GRID,),
        in_specs=[
            pl.BlockSpec((NC, R, F), lambda i: (0, i, 0)),
            pl.BlockSpec((R, F), lambda i: (i, 0)),
            pl.BlockSpec((NC, R // F, F), lambda i: (0, i, 0)),
            pl.BlockSpec((1, F), lambda i: (0, 0)),
        ],
        out_specs=pl.BlockSpec((R, F), lambda i: (i, 0)),
        out_shape=jax.ShapeDtypeStruct((NP, F), jnp.float32),
    )(s_part, h2, cnt3, b.reshape(1, F))

    return outp[:N]


# speedup vs baseline: 16.1462x; 16.1462x over previous
import jax
import jax.numpy as jnp
from jax import lax
from jax.experimental import pallas as pl
from jax.experimental.pallas import tpu as pltpu
from jax.experimental.pallas import tpu_sc as plsc

N = 10000
NP = 10240
F = 128
E = 320000
NC, NS = 2, 16
NW = NC * NS
EPW = E // NW
CHUNK = 80
NCH = EPW // CHUNK
STRIPE = NP // NS
R = 1024
GRID = NP // R


def _fill_f32(ref, n, value):
    v = jnp.full((16,), value, dtype=jnp.float32)

    def body(i, _):
        ref[pl.ds(i * 16, 16)] = v
        return 0

    lax.fori_loop(0, n // 16, body, 0)


def _fill2_f32(ref, rows, value):
    v = jnp.full((16,), value, dtype=jnp.float32)

    def body(i, _):
        for j in range(8):
            ref[i, pl.ds(j * 16, 16)] = v
        return 0

    lax.fori_loop(0, rows, body, 0)


def _hist_body(col_hbm, cnt_hbm, hist_sh, idx_v, ones_v, tmp_v):
    c = lax.axis_index("c")
    s = lax.axis_index("s")
    wid = c * NS + s

    _fill_f32(tmp_v, STRIPE, 0.0)
    pltpu.sync_copy(tmp_v, hist_sh.at[pl.ds(s * STRIPE, STRIPE)])
    _fill_f32(ones_v, CHUNK, 1.0)
    plsc.subcore_barrier()

    def body(i, _):
        base = wid * EPW + i * CHUNK
        pltpu.sync_copy(col_hbm.at[pl.ds(base, CHUNK)], idx_v)
        pltpu.sync_copy(ones_v, hist_sh.at[idx_v], add=True)
        return 0

    lax.fori_loop(0, NCH, body, 0)
    plsc.subcore_barrier()

    pltpu.sync_copy(hist_sh.at[pl.ds(s * STRIPE, STRIPE)], tmp_v)
    pltpu.sync_copy(tmp_v, cnt_hbm.at[c, pl.ds(s * STRIPE, STRIPE)])


_hist = pl.kernel(
    _hist_body,
    out_type=jax.ShapeDtypeStruct((NC, NP), jnp.float32),
    mesh=plsc.VectorSubcoreMesh(core_axis_name="c", subcore_axis_name="s"),
    scratch_types=[
        pltpu.VMEM_SHARED((NP,), jnp.float32),
        pltpu.VMEM((CHUNK,), jnp.int32),
        pltpu.VMEM((CHUNK,), jnp.float32),
        pltpu.VMEM((STRIPE,), jnp.float32),
    ],
)


def _edge_body(row_hbm, col_hbm, h2_hbm, s_hbm, acc_sh, ridx_v, cidx_v, rows_v, sem):
    c = lax.axis_index("c")
    s = lax.axis_index("s")
    wid = c * NS + s

    _fill2_f32(rows_v, CHUNK, 0.0)
    for j in range(STRIPE // CHUNK):
        pltpu.sync_copy(rows_v, acc_sh.at[pl.ds(s * STRIPE + j * CHUNK, CHUNK)])
    plsc.subcore_barrier()

    def body(i, _):
        base = wid * EPW + i * CHUNK
        pltpu.sync_copy(row_hbm.at[pl.ds(base, CHUNK)], ridx_v)
        pltpu.async_copy(h2_hbm.at[ridx_v], rows_v, sem).wait()
        pltpu.sync_copy(col_hbm.at[pl.ds(base, CHUNK)], cidx_v)
        pltpu.sync_copy(rows_v, acc_sh.at[cidx_v], add=True)
        return 0

    lax.fori_loop(0, NCH, body, 0)
    plsc.subcore_barrier()

    for j in range(STRIPE // CHUNK):
        r0 = s * STRIPE + j * CHUNK
        pltpu.sync_copy(acc_sh.at[pl.ds(r0, CHUNK)], rows_v)
        pltpu.sync_copy(rows_v, s_hbm.at[c, pl.ds(r0, CHUNK)])


_edges = pl.kernel(
    _edge_body,
    out_type=jax.ShapeDtypeStruct((NC, NP, F), jnp.float32),
    mesh=plsc.VectorSubcoreMesh(core_axis_name="c", subcore_axis_name="s"),
    scratch_types=[
        pltpu.VMEM_SHARED((NP, F), jnp.float32),
        pltpu.VMEM((CHUNK,), jnp.int32),
        pltpu.VMEM((CHUNK,), jnp.int32),
        pltpu.VMEM((CHUNK, F), jnp.float32),
        pltpu.SemaphoreType.DMA,
    ],
)


def _eye128():
    a = lax.broadcasted_iota(jnp.int32, (F, F), 0)
    b = lax.broadcasted_iota(jnp.int32, (F, F), 1)
    return (a == b).astype(jnp.float32)


def _ln_mm_body(x_ref, g_ref, bt_ref, w_ref, cnt_ref, o_ref):
    xb = x_ref[...]
    mean = jnp.mean(xb, axis=1, keepdims=True)
    xc = xb - mean
    var = jnp.mean(xc * xc, axis=1, keepdims=True)
    h = xc * lax.rsqrt(var + 1e-5) * g_ref[...] + bt_ref[...]
    h = jnp.maximum(h, 0.0)
    hw = jnp.dot(h, w_ref[...], preferred_element_type=jnp.float32)
    cb = cnt_ref[...]
    dis = lax.rsqrt(cb[0] + cb[1] + 1.0)
    eye = _eye128()
    for r in range(R // F):
        diag = eye * dis[r][None, :]
        o_ref[r * F:(r + 1) * F, :] = jnp.dot(
            diag, hw[r * F:(r + 1) * F, :], preferred_element_type=jnp.float32)


def _final_body(s_ref, h2_ref, cnt_ref, b_ref, o_ref):
    sb = s_ref[...]
    t = sb[0] + sb[1] + h2_ref[...]
    cb = cnt_ref[...]
    dis = lax.rsqrt(cb[0] + cb[1] + 1.0)
    eye = _eye128()
    bias = b_ref[...]
    for r in range(R // F):
        diag = eye * dis[r][None, :]
        o_ref[r * F:(r + 1) * F, :] = jnp.dot(
            diag, t[r * F:(r + 1) * F, :], preferred_element_type=jnp.float32) + bias


@jax.jit
def kernel(x, edge_index, gamma, beta, W, b):
    edge_index = edge_index.astype(jnp.int32)
    row = edge_index[0]
    col = edge_index[1]
    cnt = _hist(col)

    xp = jnp.zeros((NP, F), x.dtype).at[:N].set(x)
    cnt3 = cnt.reshape(NC, NP // F, F)

    h2 = pl.pallas_call(
        _ln_mm_body,
        grid=(GRID,),
        in_specs=[
            pl.BlockSpec((R, F), lambda i: (i, 0)),
            pl.BlockSpec((1, F), lambda i: (0, 0)),
            pl.BlockSpec((1, F), lambda i: (0, 0)),
            pl.BlockSpec((F, F), lambda i: (0, 0)),
            pl.BlockSpec((NC, R // F, F), lambda i: (0, i, 0)),
        ],
        out_specs=pl.BlockSpec((R, F), lambda i: (i, 0)),
        out_shape=jax.ShapeDtypeStruct((NP, F), jnp.float32),
    )(xp, gamma.reshape(1, F), beta.reshape(1, F), W, cnt3)

    s_part = _edges(row, col, h2)

    outp = pl.pallas_call(
        _final_body,
        grid=(GRID,),
        in_specs=[
            pl.BlockSpec((NC, R, F), lambda i: (0, i, 0)),
            pl.BlockSpec((R, F), lambda i: (i, 0)),
            pl.BlockSpec((NC, R // F, F), lambda i: (0, i, 0)),
            pl.BlockSpec((1, F), lambda i: (0, 0)),
        ],
        out_specs=pl.BlockSpec((R, F), lambda i: (i, 0)),
        out_shape=jax.ShapeDtypeStruct((NP, F), jnp.float32),
    )(s_part, h2, cnt3, b.reshape(1, F))

    return outp[:N]
